# baseline (speedup 1.0000x reference)
import jax
import jax.numpy as jnp
from jax import lax
from jax.experimental import pallas as pl
from jax.experimental.pallas import tpu as pltpu
from jax.experimental.pallas import tpu_sc as plsc

D = 1024
B = 4
BSC = 3
S = 4096
NC = 2
NS = 16
NW = NC * NS
SEQ_PER_W = S // NW
CHUNK = 16
NCHUNK = SEQ_PER_W // CHUNK
NTILE = NCHUNK * BSC
NVREG = CHUNK * D // 16
NTOK = 5
AHEAD = 3
TCROWS = 8


def _embed_body(idx_hbm, tok_hbm, pos_hbm, out_hbm,
                idx_v, tok0, tok1, tok2, tok3, tok4, posA, posB,
                gs0, gs1, gs2, gs3, gs4, ss0, ss1, ss2, ss3, ss4, psA, psB):
    wid = lax.axis_index("s") * NC + lax.axis_index("c")
    s0 = wid * SEQ_PER_W
    tok_b = [tok0, tok1, tok2, tok3, tok4]
    pos_b = [posA, posB]
    gsem = [gs0, gs1, gs2, gs3, gs4]
    ssem = [ss0, ss1, ss2, ss3, ss4]
    psem = [psA, psB]

    pltpu.sync_copy(idx_hbm.at[wid], idx_v)

    def start_gather(u):
        a = u % NTOK
        pltpu.async_copy(tok_hbm.at[idx_v.at[u]], tok_b[a], gsem[a])

    def wait_gather(a):
        pltpu.make_async_copy(
            pos_hbm.at[pl.ds(0, CHUNK)], tok_b[a], gsem[a]).wait()

    def start_pos(g):
        pltpu.async_copy(
            pos_hbm.at[pl.ds(s0 + g * CHUNK, CHUNK)], pos_b[g % 2],
            psem[g % 2])

    def wait_pos(pb):
        pltpu.make_async_copy(
            pos_hbm.at[pl.ds(0, CHUNK)], pos_b[pb], psem[pb]).wait()

    def start_store(u):
        a = u % NTOK
        g, b = u // BSC, u % BSC
        pltpu.async_copy(
            tok_b[a], out_hbm.at[pl.ds(b * S + s0 + g * CHUNK, CHUNK)],
            ssem[a])

    def wait_store(a):
        pltpu.make_async_copy(
            tok_b[a], out_hbm.at[pl.ds(0, CHUNK)], ssem[a]).wait()

    def compute(a, pb):
        tv, pv = tok_b[a], pos_b[pb]

        @plsc.parallel_loop(0, NVREG, unroll=8)
        def _(t):
            r = lax.shift_right_logical(t, 6)
            col = lax.shift_left(lax.bitwise_and(t, 63), 4)
            sl = pl.ds(pl.multiple_of(col, 16), 16)
            plsc.addupdate(tv.at[r, sl], pv[r, sl])

    start_pos(0)
    start_pos(1)
    for u in range(AHEAD):
        start_gather(u)

    for u in range(NTILE):
        a = u % NTOK
        g, b = u // BSC, u % BSC
        wait_gather(a)
        if b == 0:
            wait_pos(g % 2)
        compute(a, g % 2)
        start_store(u)
        if b == BSC - 1 and g + 2 < NCHUNK:
            start_pos(g + 2)
        un = u + AHEAD
        if un < NTILE:
            an = un % NTOK
            if un >= NTOK:
                wait_store(an)
            start_gather(un)

    for a in range(NTOK):
        wait_store(a)


def _tc_body(idx_ref, tbl, pos_blk, out_blk, gbuf, sems):
    r = pl.program_id(0)
    nsteps = pl.num_programs(0)

    def fire(step, slot):
        for j in range(TCROWS):
            row = idx_ref[step * TCROWS + j]
            pltpu.make_async_copy(
                tbl.at[pl.ds(row * D, D)], gbuf.at[slot, pl.ds(j * D, D)],
                sems.at[slot]).start()

    def drain(slot):
        pltpu.make_async_copy(
            tbl.at[pl.ds(0, TCROWS * D)], gbuf.at[slot],
            sems.at[slot]).wait()

    @pl.when(r == 0)
    def _():
        fire(0, 0)

    @pl.when(r + 1 < nsteps)
    def _():
        fire(r + 1, (r + 1) % 2)

    cur = r % 2
    drain(cur)
    out_blk[...] = gbuf[cur] + pos_blk[...]


def kernel(idx, token_embd_table, position_embd_table):
    batch, seq = idx.shape
    idx_sc = jnp.transpose(
        idx[:BSC].reshape(BSC, NW, NCHUNK, CHUNK), (1, 2, 0, 3))
    idx_sc = idx_sc.reshape(NW, NTILE, CHUNK).astype(jnp.int32)
    mesh = plsc.VectorSubcoreMesh(core_axis_name="c", subcore_axis_name="s")
    k_sc = pl.kernel(
        _embed_body,
        mesh=mesh,
        out_type=jax.ShapeDtypeStruct((BSC * S, D), jnp.float32),
        scratch_types=[
            pltpu.VMEM((NTILE, CHUNK), jnp.int32),
        ] + [pltpu.VMEM((CHUNK, D), jnp.float32)] * (NTOK + 2)
          + [pltpu.SemaphoreType.DMA] * (2 * NTOK + 2),
    )
    out_sc = k_sc(idx_sc, token_embd_table, position_embd_table)

    idx_tc = idx[BSC].reshape(S).astype(jnp.int32)
    tbl_flat = token_embd_table.reshape(token_embd_table.size)
    pos_flat = position_embd_table.reshape(position_embd_table.size)
    out_tc = pl.pallas_call(
        _tc_body,
        grid_spec=pltpu.PrefetchScalarGridSpec(
            num_scalar_prefetch=1,
            grid=(S // TCROWS,),
            in_specs=[
                pl.BlockSpec(memory_space=pl.ANY),
                pl.BlockSpec((TCROWS * D,), lambda r, *_: (r,)),
            ],
            out_specs=pl.BlockSpec((TCROWS * D,), lambda r, *_: (r,)),
            scratch_shapes=[
                pltpu.VMEM((2, TCROWS * D), jnp.float32),
                pltpu.SemaphoreType.DMA((2,)),
            ],
        ),
        out_shape=jax.ShapeDtypeStruct((S * D,), jnp.float32),
    )(idx_tc, tbl_flat, pos_flat)

    out = jnp.concatenate([out_sc.reshape(BSC * S * D), out_tc])
    return out.reshape(batch, seq, D)


# speedup vs baseline: 9.8495x; 9.8495x over previous
import jax
import jax.numpy as jnp
from jax import lax
from jax.experimental import pallas as pl
from jax.experimental.pallas import tpu as pltpu
from jax.experimental.pallas import tpu_sc as plsc

D = 1024
B = 4
S = 4096
NC = 2
NS = 16
NW = NC * NS
SEQ_PER_W = S // NW
CHUNK = 16
NCHUNK = SEQ_PER_W // CHUNK
NTILE = NCHUNK * B
NVREG = CHUNK * D // 16
NTOK = 5
AHEAD = 3


def _embed_body(idx_hbm, tok_hbm, pos_hbm, out_hbm,
                idx_v, tok0, tok1, tok2, tok3, tok4, posA, posB,
                gs0, gs1, gs2, gs3, gs4, ss0, ss1, ss2, ss3, ss4, psA, psB):
    wid = lax.axis_index("s") * NC + lax.axis_index("c")
    s0 = wid * SEQ_PER_W
    tok_b = [tok0, tok1, tok2, tok3, tok4]
    pos_b = [posA, posB]
    gsem = [gs0, gs1, gs2, gs3, gs4]
    ssem = [ss0, ss1, ss2, ss3, ss4]
    psem = [psA, psB]

    pltpu.sync_copy(idx_hbm.at[wid], idx_v)

    def start_gather(u):
        a = u % NTOK
        g, b = u // B, u % B
        pltpu.async_copy(tok_hbm.at[idx_v.at[u]], tok_b[a], gsem[a])

    def wait_gather(a):
        pltpu.make_async_copy(
            pos_hbm.at[pl.ds(0, CHUNK)], tok_b[a], gsem[a]).wait()

    def start_pos(g):
        pltpu.async_copy(
            pos_hbm.at[pl.ds(s0 + g * CHUNK, CHUNK)], pos_b[g % 2],
            psem[g % 2])

    def wait_pos(pb):
        pltpu.make_async_copy(
            pos_hbm.at[pl.ds(0, CHUNK)], pos_b[pb], psem[pb]).wait()

    def start_store(u):
        a = u % NTOK
        g, b = u // B, u % B
        pltpu.async_copy(
            tok_b[a], out_hbm.at[pl.ds(b * S + s0 + g * CHUNK, CHUNK)],
            ssem[a])

    def wait_store(a):
        pltpu.make_async_copy(
            tok_b[a], out_hbm.at[pl.ds(0, CHUNK)], ssem[a]).wait()

    def compute(a, pb):
        tv, pv = tok_b[a], pos_b[pb]

        @plsc.parallel_loop(0, NVREG, unroll=16)
        def _(t):
            r = lax.shift_right_logical(t, 6)
            col = lax.shift_left(lax.bitwise_and(t, 63), 4)
            sl = pl.ds(pl.multiple_of(col, 16), 16)
            plsc.addupdate(tv.at[r, sl], pv[r, sl])

    start_pos(0)
    start_pos(1)
    for u in range(AHEAD):
        start_gather(u)

    for u in range(NTILE):
        a = u % NTOK
        g, b = u // B, u % B
        wait_gather(a)
        if b == 0:
            wait_pos(g % 2)
        compute(a, g % 2)
        start_store(u)
        if b == B - 1 and g + 2 < NCHUNK:
            start_pos(g + 2)
        un = u + AHEAD
        if un < NTILE:
            an = un % NTOK
            if un >= NTOK:
                wait_store(an)
            start_gather(un)

    for a in range(NTOK):
        wait_store(a)


def kernel(idx, token_embd_table, position_embd_table):
    batch, seq = idx.shape
    idx_r = jnp.transpose(idx.reshape(batch, NW, NCHUNK, CHUNK), (1, 2, 0, 3))
    idx_r = idx_r.reshape(NW, NTILE, CHUNK).astype(jnp.int32)
    mesh = plsc.VectorSubcoreMesh(core_axis_name="c", subcore_axis_name="s")
    k = pl.kernel(
        _embed_body,
        mesh=mesh,
        out_type=jax.ShapeDtypeStruct((batch * seq, D), jnp.float32),
        scratch_types=[
            pltpu.VMEM((NTILE, CHUNK), jnp.int32),
        ] + [pltpu.VMEM((CHUNK, D), jnp.float32)] * (NTOK + 2)
          + [pltpu.SemaphoreType.DMA] * (2 * NTOK + 2),
    )
    out = k(idx_r, token_embd_table, position_embd_table)
    return out.reshape(batch, seq, D)


# speedup vs baseline: 10.1126x; 1.0267x over previous
import jax
import jax.numpy as jnp
from jax import lax
from jax.experimental import pallas as pl
from jax.experimental.pallas import tpu as pltpu
from jax.experimental.pallas import tpu_sc as plsc

D = 1024
B = 4
S = 4096
NC = 2
NS = 16
NW = NC * NS
SEQ_PER_W = S // NW
CHUNK = 16
NCHUNK = SEQ_PER_W // CHUNK
NTILE = NCHUNK * B
NVREG = CHUNK * D // 16
NTOK = 5
AHEAD = 3


def _embed_body(idx_hbm, tok_hbm, pos_hbm, out_hbm,
                idx_v, tok0, tok1, tok2, tok3, tok4, posA, posB,
                gs0, gs1, gs2, gs3, gs4, ss0, ss1, ss2, ss3, ss4, psA, psB):
    wid = lax.axis_index("s") * NC + lax.axis_index("c")
    s0 = wid * SEQ_PER_W
    tok_b = [tok0, tok1, tok2, tok3, tok4]
    pos_b = [posA, posB]
    gsem = [gs0, gs1, gs2, gs3, gs4]
    ssem = [ss0, ss1, ss2, ss3, ss4]
    psem = [psA, psB]

    pltpu.sync_copy(idx_hbm.at[wid], idx_v)

    def start_gather(u):
        a = u % NTOK
        g, b = u // B, u % B
        pltpu.async_copy(tok_hbm.at[idx_v.at[u]], tok_b[a], gsem[a])

    def wait_gather(a):
        pltpu.make_async_copy(
            pos_hbm.at[pl.ds(0, CHUNK)], tok_b[a], gsem[a]).wait()

    def start_pos(g):
        pltpu.async_copy(
            pos_hbm.at[pl.ds(s0 + g * CHUNK, CHUNK)], pos_b[g % 2],
            psem[g % 2])

    def wait_pos(pb):
        pltpu.make_async_copy(
            pos_hbm.at[pl.ds(0, CHUNK)], pos_b[pb], psem[pb]).wait()

    def start_store(u):
        a = u % NTOK
        g, b = u // B, u % B
        pltpu.async_copy(
            tok_b[a], out_hbm.at[pl.ds(b * S + s0 + g * CHUNK, CHUNK)],
            ssem[a])

    def wait_store(a):
        pltpu.make_async_copy(
            tok_b[a], out_hbm.at[pl.ds(0, CHUNK)], ssem[a]).wait()

    def compute(a, pb):
        tv, pv = tok_b[a], pos_b[pb]

        @plsc.parallel_loop(0, NVREG, unroll=8)
        def _(t):
            r = lax.shift_right_logical(t, 6)
            col = lax.shift_left(lax.bitwise_and(t, 63), 4)
            sl = pl.ds(pl.multiple_of(col, 16), 16)
            plsc.addupdate(tv.at[r, sl], pv[r, sl])

    start_pos(0)
    start_pos(1)
    for u in range(AHEAD):
        start_gather(u)

    for u in range(NTILE):
        a = u % NTOK
        g, b = u // B, u % B
        wait_gather(a)
        if b == 0:
            wait_pos(g % 2)
        compute(a, g % 2)
        start_store(u)
        if b == B - 1 and g + 2 < NCHUNK:
            start_pos(g + 2)
        un = u + AHEAD
        if un < NTILE:
            an = un % NTOK
            if un >= NTOK:
                wait_store(an)
            start_gather(un)

    for a in range(NTOK):
        wait_store(a)


def kernel(idx, token_embd_table, position_embd_table):
    batch, seq = idx.shape
    idx_r = jnp.transpose(idx.reshape(batch, NW, NCHUNK, CHUNK), (1, 2, 0, 3))
    idx_r = idx_r.reshape(NW, NTILE, CHUNK).astype(jnp.int32)
    mesh = plsc.VectorSubcoreMesh(core_axis_name="c", subcore_axis_name="s")
    k = pl.kernel(
        _embed_body,
        mesh=mesh,
        out_type=jax.ShapeDtypeStruct((batch * seq, D), jnp.float32),
        scratch_types=[
            pltpu.VMEM((NTILE, CHUNK), jnp.int32),
        ] + [pltpu.VMEM((CHUNK, D), jnp.float32)] * (NTOK + 2)
          + [pltpu.SemaphoreType.DMA] * (2 * NTOK + 2),
    )
    out = k(idx_r, token_embd_table, position_embd_table)
    return out.reshape(batch, seq, D)


# speedup vs baseline: 10.1307x; 1.0018x over previous
import jax
import jax.numpy as jnp
from jax import lax
from jax.experimental import pallas as pl
from jax.experimental.pallas import tpu as pltpu
from jax.experimental.pallas import tpu_sc as plsc

D = 1024
B = 4
S = 4096
NC = 2
NS = 16
NW = NC * NS
SEQ_PER_W = S // NW
CHUNK = 16
NCHUNK = SEQ_PER_W // CHUNK
NTILE = NCHUNK * B
NVREG = CHUNK * D // 16
NTOK = 5
AHEAD = 3


def _embed_body(idx_hbm, tok_hbm, pos_hbm, out_hbm,
                idx_v, tok0, tok1, tok2, tok3, tok4, posA, posB,
                gs0, gs1, gs2, gs3, gs4, ss0, ss1, ss2, ss3, ss4, psA, psB):
    wid = lax.axis_index("s") * NC + lax.axis_index("c")
    s0 = wid * SEQ_PER_W
    tok_b = [tok0, tok1, tok2, tok3, tok4]
    pos_b = [posA, posB]
    gsem = [gs0, gs1, gs2, gs3, gs4]
    ssem = [ss0, ss1, ss2, ss3, ss4]
    psem = [psA, psB]

    pltpu.sync_copy(idx_hbm.at[wid], idx_v)

    def start_gather(u):
        a = u % NTOK
        pltpu.async_copy(tok_hbm.at[idx_v.at[u]], tok_b[a], gsem[a])

    def wait_gather(a):
        pltpu.make_async_copy(
            pos_hbm.at[pl.ds(0, CHUNK)], tok_b[a], gsem[a]).wait()

    def start_pos(g):
        pltpu.async_copy(
            pos_hbm.at[pl.ds(s0 + g * CHUNK, CHUNK)], pos_b[g % 2],
            psem[g % 2])

    def wait_pos(pb):
        pltpu.make_async_copy(
            pos_hbm.at[pl.ds(0, CHUNK)], pos_b[pb], psem[pb]).wait()

    def start_store(u):
        a = u % NTOK
        g, b = u // B, u % B
        pltpu.async_copy(
            tok_b[a], out_hbm.at[pl.ds(b * S + s0 + g * CHUNK, CHUNK)],
            ssem[a])

    def wait_store(a):
        pltpu.make_async_copy(
            tok_b[a], out_hbm.at[pl.ds(0, CHUNK)], ssem[a]).wait()

    def compute(a, pb):
        tv, pv = tok_b[a], pos_b[pb]

        @plsc.parallel_loop(0, NVREG, unroll=8)
        def _(t):
            r = lax.shift_right_logical(t, 6)
            col = lax.shift_left(lax.bitwise_and(t, 63), 4)
            sl = pl.ds(pl.multiple_of(col, 16), 16)
            plsc.addupdate(tv.at[r, sl], pv[r, sl])

    start_pos(0)
    start_pos(1)
    for u in range(AHEAD):
        start_gather(u)

    for u in range(NTILE):
        a = u % NTOK
        g, b = u // B, u % B
        wait_gather(a)
        if b == 0:
            wait_pos(g % 2)
        compute(a, g % 2)
        start_store(u)
        if b == B - 1 and g + 2 < NCHUNK:
            start_pos(g + 2)
        un = u + AHEAD
        if un < NTILE:
            an = un % NTOK
            if un >= NTOK:
                wait_store(an)
            start_gather(un)

    for a in range(NTOK):
        wait_store(a)


def kernel(idx, token_embd_table, position_embd_table):
    batch, seq = idx.shape
    idx_r = jnp.transpose(idx.reshape(batch, NW, NCHUNK, CHUNK), (1, 2, 0, 3))
    idx_r = idx_r.reshape(NW, NTILE, CHUNK).astype(jnp.int32)
    mesh = plsc.VectorSubcoreMesh(core_axis_name="c", subcore_axis_name="s")
    k = pl.kernel(
        _embed_body,
        mesh=mesh,
        out_type=jax.ShapeDtypeStruct((batch * seq, D), jnp.float32),
        scratch_types=[
            pltpu.VMEM((NTILE, CHUNK), jnp.int32),
        ] + [pltpu.VMEM((CHUNK, D), jnp.float32)] * (NTOK + 2)
          + [pltpu.SemaphoreType.DMA] * (2 * NTOK + 2),
    )
    out = k(idx_r, token_embd_table, position_embd_table)
    return out.reshape(batch, seq, D)
